# baseline (speedup 1.0000x reference)
import functools

import jax
import jax.numpy as jnp
from jax.experimental import pallas as pl
from jax.experimental.pallas import tpu as pltpu

B, K, D, E, P = 128, 36, 2048, 1024, 5
M = B * K
TILE = 16 * K
NTILES = M // TILE

_PARALLEL = pltpu.CompilerParams(dimension_semantics=("parallel",))


def _stage1_body(x_ref, bb_ref, w1ig_ref, w1in_ref, w1eg_ref, w1en_ref,
                 gb1_ref, nb1_ref, h_ref):
    X = x_ref[...]
    bb = bb_ref[...]
    area = (bb[:, 2:3] - bb[:, 0:1]) * (bb[:, 3:4] - bb[:, 1:2])
    extras = (jnp.concatenate(
        [bb[:, 0:4], area, jnp.zeros((TILE, 3), jnp.float32)], axis=1)
        * 0.1).astype(jnp.bfloat16)
    hg = jnp.dot(X, w1ig_ref[...], preferred_element_type=jnp.float32)
    hg = hg + jnp.dot(extras, w1eg_ref[...],
                      preferred_element_type=jnp.float32) + gb1_ref[...]
    hn = jnp.dot(X, w1in_ref[...], preferred_element_type=jnp.float32)
    hn = hn + jnp.dot(extras, w1en_ref[...],
                      preferred_element_type=jnp.float32) + nb1_ref[...]
    h_ref[:, :D] = jnp.maximum(hg, 0.0).astype(jnp.bfloat16)
    h_ref[:, D:] = jnp.maximum(hn, 0.0).astype(jnp.bfloat16)


def _stage2_body(h_ref, x_ref, r_ref, w2n_ref, nb2_ref, gw2_ref, gb2_ref,
                 out_ref):
    h = h_ref[...]
    hg = h[:, :D]
    hn = h[:, D:]
    v = jnp.dot(hn, w2n_ref[...], preferred_element_type=jnp.float32)
    v = v + nb2_ref[...]
    gate = jnp.dot(hg, gw2_ref[...],
                   preferred_element_type=jnp.float32)[:, 0:1] + gb2_ref[...]
    m = jax.nn.sigmoid(gate)
    vm = (m * v).astype(jnp.bfloat16)

    R = r_ref[...]
    iu = jax.lax.broadcasted_iota(jnp.int32, (K, K), 0)
    ju = jax.lax.broadcasted_iota(jnp.int32, (K, K), 1)
    upper = (iu <= ju).astype(jnp.bfloat16)
    cs = jnp.dot(R.astype(jnp.bfloat16), upper,
                 preferred_element_type=jnp.float32)
    sel = jnp.where((R == 1.0) & (cs <= float(P)), 1.0, 0.0)
    deficit = float(P) - jnp.minimum(cs[:, K - 1:K], float(P))

    jg = jax.lax.broadcasted_iota(jnp.int32, (K, TILE), 0)
    cg = jax.lax.broadcasted_iota(jnp.int32, (K, TILE), 1)
    G = (cg % K == jg).astype(jnp.bfloat16)
    W = jnp.dot(sel.astype(jnp.bfloat16), G,
                preferred_element_type=jnp.float32)
    ri = jax.lax.broadcasted_iota(jnp.int32, (TILE, TILE), 0)
    ci = jax.lax.broadcasted_iota(jnp.int32, (TILE, TILE), 1)
    W = jnp.where((ri // K) == (ci // K), W, 0.0)
    W = W + jnp.where(ri == ci, deficit, 0.0)

    agg = jnp.dot(W.astype(jnp.bfloat16), vm,
                  preferred_element_type=jnp.float32)
    norm = jnp.sqrt(jnp.sum(agg * agg, axis=1, keepdims=True)) + 1e-8
    out_ref[...] = x_ref[...] + agg / norm


def _stage3_body(x2_ref, w3_ref, b3_ref, w4_ref, b4_ref, o_ref):
    hm = jnp.dot(x2_ref[...].astype(jnp.bfloat16), w3_ref[...],
                 preferred_element_type=jnp.float32)
    hm = jnp.maximum(hm + b3_ref[...], 0.0).astype(jnp.bfloat16)
    emb = jnp.dot(hm, w4_ref[...], preferred_element_type=jnp.float32)
    emb = emb + b4_ref[...]
    norm = jnp.sqrt(jnp.sum(emb * emb, axis=1, keepdims=True)) + 1e-8
    o_ref[...] = emb / norm


def _prep_body(x_ref, gw1_ref, nw1_ref, nw2_ref, mw1_ref, mw2_ref, gw2_ref,
               xb_ref, w1ig_ref, w1in_ref, w1eg_ref, w1en_ref, w2n_ref,
               w3_ref, w4_ref, gw2c_ref):
    xb_ref[...] = x_ref[...].astype(jnp.bfloat16)
    w1ig_ref[...] = gw1_ref[:, :D].astype(jnp.bfloat16).T
    w1in_ref[...] = nw1_ref[:, :D].astype(jnp.bfloat16).T
    pad = jnp.zeros((3, gw1_ref.shape[0]), jnp.bfloat16)
    w1eg_ref[...] = jnp.concatenate(
        [gw1_ref[:, D:].astype(jnp.bfloat16).T, pad], axis=0)
    w1en_ref[...] = jnp.concatenate(
        [nw1_ref[:, D:].astype(jnp.bfloat16).T, pad], axis=0)
    w2n_ref[...] = nw2_ref[...].astype(jnp.bfloat16).T
    w3_ref[...] = mw1_ref[...].astype(jnp.bfloat16).T
    w4_ref[...] = mw2_ref[...].astype(jnp.bfloat16).T
    gw2c = gw2_ref[...].astype(jnp.bfloat16).T
    gw2c_ref[...] = jnp.concatenate(
        [gw2c, jnp.zeros((gw2c.shape[0], 127), jnp.bfloat16)], axis=1)


def _prep_weights(X, gw1, nw1, nw2, mw1, mw2, gw2, interpret):
    g = 8
    rx = M // g
    r1 = D // g
    r2 = E // g
    return pl.pallas_call(
        _prep_body,
        grid=(g,),
        in_specs=[pl.BlockSpec((rx, D), lambda i: (i, 0)),
                  pl.BlockSpec((r1, D + 5), lambda i: (i, 0)),
                  pl.BlockSpec((r1, D + 5), lambda i: (i, 0)),
                  pl.BlockSpec((r1, D), lambda i: (i, 0)),
                  pl.BlockSpec((r1, D), lambda i: (i, 0)),
                  pl.BlockSpec((r2, D), lambda i: (i, 0)),
                  pl.BlockSpec((1, r1), lambda i: (0, i))],
        out_specs=[pl.BlockSpec((rx, D), lambda i: (i, 0)),
                   pl.BlockSpec((D, r1), lambda i: (0, i)),
                   pl.BlockSpec((D, r1), lambda i: (0, i)),
                   pl.BlockSpec((8, r1), lambda i: (0, i)),
                   pl.BlockSpec((8, r1), lambda i: (0, i)),
                   pl.BlockSpec((D, r1), lambda i: (0, i)),
                   pl.BlockSpec((D, r1), lambda i: (0, i)),
                   pl.BlockSpec((D, r2), lambda i: (0, i)),
                   pl.BlockSpec((r1, 128), lambda i: (i, 0))],
        out_shape=[jax.ShapeDtypeStruct((M, D), jnp.bfloat16),
                   jax.ShapeDtypeStruct((D, D), jnp.bfloat16),
                   jax.ShapeDtypeStruct((D, D), jnp.bfloat16),
                   jax.ShapeDtypeStruct((8, D), jnp.bfloat16),
                   jax.ShapeDtypeStruct((8, D), jnp.bfloat16),
                   jax.ShapeDtypeStruct((D, D), jnp.bfloat16),
                   jax.ShapeDtypeStruct((D, D), jnp.bfloat16),
                   jax.ShapeDtypeStruct((D, E), jnp.bfloat16),
                   jax.ShapeDtypeStruct((D, 128), jnp.bfloat16)],
        compiler_params=_PARALLEL,
        interpret=interpret,
    )(X, gw1, nw1, nw2, mw1, mw2, gw2)


def _row_spec(n):
    return pl.BlockSpec((TILE, n), lambda i: (i, 0))


def _full_spec(m, n):
    return pl.BlockSpec((m, n), lambda i: (0, 0))


@functools.partial(jax.jit, static_argnames=("interpret",))
def _run(images, bboxes, img_range, gw1, gb1, gw2, gb2, nw1, nb1, nw2, nb2,
         mw1, mb1, mw2, mb2, interpret=False):
    X = images.reshape(M, D)
    bb = jnp.pad(bboxes.reshape(M, 4), ((0, 0), (0, 4)))
    R = img_range.reshape(M, K)

    xb, w1ig, w1in, w1eg, w1en, w2n, w3, w4, gw2c = _prep_weights(
        X, gw1, nw1, nw2, mw1, mw2, gw2, interpret)
    gb1r = gb1[None, :]
    nb1r = nb1[None, :]
    nb2r = nb2[None, :]
    gb2r = gb2[None, :]
    b3 = mb1[None, :]
    b4 = mb2[None, :]

    h = pl.pallas_call(
        _stage1_body,
        grid=(NTILES,),
        in_specs=[_row_spec(D), _row_spec(8), _full_spec(D, D),
                  _full_spec(D, D), _full_spec(8, D), _full_spec(8, D),
                  _full_spec(1, D), _full_spec(1, D)],
        out_specs=_row_spec(2 * D),
        out_shape=jax.ShapeDtypeStruct((M, 2 * D), jnp.bfloat16),
        compiler_params=_PARALLEL,
        interpret=interpret,
    )(xb, bb, w1ig, w1in, w1eg, w1en, gb1r, nb1r)

    x2 = pl.pallas_call(
        _stage2_body,
        grid=(NTILES,),
        in_specs=[_row_spec(2 * D), _row_spec(D), _row_spec(K),
                  _full_spec(D, D), _full_spec(1, D), _full_spec(D, 128),
                  _full_spec(1, 1)],
        out_specs=_row_spec(D),
        out_shape=jax.ShapeDtypeStruct((M, D), jnp.float32),
        compiler_params=_PARALLEL,
        interpret=interpret,
    )(h, X, R, w2n, nb2r, gw2c, gb2r)

    emb = pl.pallas_call(
        _stage3_body,
        grid=(NTILES,),
        in_specs=[_row_spec(D), _full_spec(D, D), _full_spec(1, D),
                  _full_spec(D, E), _full_spec(1, E)],
        out_specs=_row_spec(E),
        out_shape=jax.ShapeDtypeStruct((M, E), jnp.float32),
        compiler_params=_PARALLEL,
        interpret=interpret,
    )(x2, w3, b3, w4, b4)

    return emb.reshape(B, K, E)


def kernel(images, bboxes, img_range, gw1, gb1, gw2, gb2, nw1, nb1, nw2, nb2,
           mw1, mb1, mw2, mb2):
    return _run(images, bboxes, img_range, gw1, gb1, gw2, gb2, nw1, nb1,
                nw2, nb2, mw1, mb1, mw2, mb2)


# speedup vs baseline: 1.0064x; 1.0064x over previous
import functools

import jax
import jax.numpy as jnp
from jax.experimental import pallas as pl
from jax.experimental.pallas import tpu as pltpu

B, K, D, E, P = 128, 36, 2048, 1024, 5
M = B * K
TILE = 512
TILEA = 16 * K

_PARALLEL = pltpu.CompilerParams(dimension_semantics=("parallel",))


def _stage1_body(x_ref, bb_ref, w1ig_ref, w1in_ref, w1eg_ref, w1en_ref,
                 gb1_ref, nb1_ref, h_ref):
    X = x_ref[...].astype(jnp.bfloat16)
    bb = bb_ref[...]
    area = (bb[:, 2:3] - bb[:, 0:1]) * (bb[:, 3:4] - bb[:, 1:2])
    extras = (jnp.concatenate(
        [bb, area, jnp.zeros((TILE, 3), jnp.float32)], axis=1)
        * 0.1).astype(jnp.bfloat16)
    hg = jnp.dot(X, w1ig_ref[...], preferred_element_type=jnp.float32)
    hn = jnp.dot(X, w1in_ref[...], preferred_element_type=jnp.float32)
    hg = hg + jnp.dot(extras, w1eg_ref[...],
                      preferred_element_type=jnp.float32) + gb1_ref[...]
    hn = hn + jnp.dot(extras, w1en_ref[...],
                      preferred_element_type=jnp.float32) + nb1_ref[...]
    h_ref[:, :D] = jnp.maximum(hg, 0.0).astype(jnp.bfloat16)
    h_ref[:, D:] = jnp.maximum(hn, 0.0).astype(jnp.bfloat16)


def _stage2a_body(h_ref, w2n_ref, nb2_ref, gw2c_ref, gb2_ref, vm_ref):
    h = h_ref[...]
    hg = h[:, :D]
    hn = h[:, D:]
    v = jnp.dot(hn, w2n_ref[...], preferred_element_type=jnp.float32)
    v = v + nb2_ref[...]
    gate = jnp.dot(hg, gw2c_ref[...],
                   preferred_element_type=jnp.float32)[:, 0:1] + gb2_ref[...]
    m = jax.nn.sigmoid(gate)
    vm_ref[...] = (m * v).astype(jnp.bfloat16)


def _stage2b_body(vm_ref, x_ref, r_ref, out_ref):
    vm = vm_ref[...]
    R = r_ref[...]
    iu = jax.lax.broadcasted_iota(jnp.int32, (K, K), 0)
    ju = jax.lax.broadcasted_iota(jnp.int32, (K, K), 1)
    upper = (iu <= ju).astype(jnp.bfloat16)
    cs = jnp.dot(R.astype(jnp.bfloat16), upper,
                 preferred_element_type=jnp.float32)
    sel = jnp.where((R == 1.0) & (cs <= float(P)), 1.0, 0.0)
    deficit = float(P) - jnp.minimum(cs[:, K - 1:K], float(P))

    jg = jax.lax.broadcasted_iota(jnp.int32, (K, TILEA), 0)
    cg = jax.lax.broadcasted_iota(jnp.int32, (K, TILEA), 1)
    G = (cg % K == jg).astype(jnp.bfloat16)
    W = jnp.dot(sel.astype(jnp.bfloat16), G,
                preferred_element_type=jnp.float32)
    ri = jax.lax.broadcasted_iota(jnp.int32, (TILEA, TILEA), 0)
    ci = jax.lax.broadcasted_iota(jnp.int32, (TILEA, TILEA), 1)
    W = jnp.where((ri // K) == (ci // K), W, 0.0)
    W = W + jnp.where(ri == ci, deficit, 0.0)

    agg = jnp.dot(W.astype(jnp.bfloat16), vm,
                  preferred_element_type=jnp.float32)
    norm = jnp.sqrt(jnp.sum(agg * agg, axis=1, keepdims=True)) + 1e-8
    out_ref[...] = x_ref[...] + agg / norm


def _stage3_body(x2_ref, w3_ref, b3_ref, w4_ref, b4_ref, o_ref):
    hm = jnp.dot(x2_ref[...].astype(jnp.bfloat16), w3_ref[...],
                 preferred_element_type=jnp.float32)
    hm = jnp.maximum(hm + b3_ref[...], 0.0).astype(jnp.bfloat16)
    emb = jnp.dot(hm, w4_ref[...], preferred_element_type=jnp.float32)
    emb = emb + b4_ref[...]
    norm = jnp.sqrt(jnp.sum(emb * emb, axis=1, keepdims=True)) + 1e-8
    o_ref[...] = emb / norm


def _prep_body(gw1_ref, nw1_ref, nw2_ref, mw1_ref, mw2_ref, gw2_ref,
               w1ig_ref, w1in_ref, w1eg_ref, w1en_ref, w2n_ref,
               w3_ref, w4_ref, gw2c_ref):
    w1ig_ref[...] = gw1_ref[:, :D].astype(jnp.bfloat16).T
    w1in_ref[...] = nw1_ref[:, :D].astype(jnp.bfloat16).T
    pad = jnp.zeros((3, gw1_ref.shape[0]), jnp.bfloat16)
    w1eg_ref[...] = jnp.concatenate(
        [gw1_ref[:, D:].astype(jnp.bfloat16).T, pad], axis=0)
    w1en_ref[...] = jnp.concatenate(
        [nw1_ref[:, D:].astype(jnp.bfloat16).T, pad], axis=0)
    w2n_ref[...] = nw2_ref[...].astype(jnp.bfloat16).T
    w3_ref[...] = mw1_ref[...].astype(jnp.bfloat16).T
    w4_ref[...] = mw2_ref[...].astype(jnp.bfloat16).T
    gw2c = gw2_ref[...].astype(jnp.bfloat16).T
    gw2c_ref[...] = jnp.concatenate(
        [gw2c, jnp.zeros((gw2c.shape[0], 127), jnp.bfloat16)], axis=1)


def _prep_weights(gw1, nw1, nw2, mw1, mw2, gw2, interpret):
    g = 8
    r1 = D // g
    r2 = E // g
    return pl.pallas_call(
        _prep_body,
        grid=(g,),
        in_specs=[pl.BlockSpec((r1, D + 5), lambda i: (i, 0)),
                  pl.BlockSpec((r1, D + 5), lambda i: (i, 0)),
                  pl.BlockSpec((r1, D), lambda i: (i, 0)),
                  pl.BlockSpec((r1, D), lambda i: (i, 0)),
                  pl.BlockSpec((r2, D), lambda i: (i, 0)),
                  pl.BlockSpec((1, r1), lambda i: (0, i))],
        out_specs=[pl.BlockSpec((D, r1), lambda i: (0, i)),
                   pl.BlockSpec((D, r1), lambda i: (0, i)),
                   pl.BlockSpec((8, r1), lambda i: (0, i)),
                   pl.BlockSpec((8, r1), lambda i: (0, i)),
                   pl.BlockSpec((D, r1), lambda i: (0, i)),
                   pl.BlockSpec((D, r1), lambda i: (0, i)),
                   pl.BlockSpec((D, r2), lambda i: (0, i)),
                   pl.BlockSpec((r1, 128), lambda i: (i, 0))],
        out_shape=[jax.ShapeDtypeStruct((D, D), jnp.bfloat16),
                   jax.ShapeDtypeStruct((D, D), jnp.bfloat16),
                   jax.ShapeDtypeStruct((8, D), jnp.bfloat16),
                   jax.ShapeDtypeStruct((8, D), jnp.bfloat16),
                   jax.ShapeDtypeStruct((D, D), jnp.bfloat16),
                   jax.ShapeDtypeStruct((D, D), jnp.bfloat16),
                   jax.ShapeDtypeStruct((D, E), jnp.bfloat16),
                   jax.ShapeDtypeStruct((D, 128), jnp.bfloat16)],
        compiler_params=_PARALLEL,
        interpret=interpret,
    )(gw1, nw1, nw2, mw1, mw2, gw2)


def _row_spec(n, t=TILE):
    return pl.BlockSpec((t, n), lambda i: (i, 0))


def _full_spec(m, n):
    return pl.BlockSpec((m, n), lambda i: (0, 0))


@functools.partial(jax.jit, static_argnames=("interpret",))
def _run(images, bboxes, img_range, gw1, gb1, gw2, gb2, nw1, nb1, nw2, nb2,
         mw1, mb1, mw2, mb2, interpret=False):
    X = images.reshape(M, D)
    bb = bboxes.reshape(M, 4)
    R = img_range.reshape(M, K)

    w1ig, w1in, w1eg, w1en, w2n, w3, w4, gw2c = _prep_weights(
        gw1, nw1, nw2, mw1, mw2, gw2, interpret)
    gb1r = gb1[None, :]
    nb1r = nb1[None, :]
    nb2r = nb2[None, :]
    gb2r = gb2[None, :]
    b3 = mb1[None, :]
    b4 = mb2[None, :]

    h = pl.pallas_call(
        _stage1_body,
        grid=(M // TILE,),
        in_specs=[_row_spec(D), _row_spec(4), _full_spec(D, D),
                  _full_spec(D, D), _full_spec(8, D), _full_spec(8, D),
                  _full_spec(1, D), _full_spec(1, D)],
        out_specs=_row_spec(2 * D),
        out_shape=jax.ShapeDtypeStruct((M, 2 * D), jnp.bfloat16),
        compiler_params=_PARALLEL,
        interpret=interpret,
    )(X, bb, w1ig, w1in, w1eg, w1en, gb1r, nb1r)

    vm = pl.pallas_call(
        _stage2a_body,
        grid=(M // TILE,),
        in_specs=[_row_spec(2 * D), _full_spec(D, D), _full_spec(1, D),
                  _full_spec(D, 128), _full_spec(1, 1)],
        out_specs=_row_spec(D),
        out_shape=jax.ShapeDtypeStruct((M, D), jnp.bfloat16),
        compiler_params=_PARALLEL,
        interpret=interpret,
    )(h, w2n, nb2r, gw2c, gb2r)

    x2 = pl.pallas_call(
        _stage2b_body,
        grid=(M // TILEA,),
        in_specs=[_row_spec(D, TILEA), _row_spec(D, TILEA),
                  _row_spec(K, TILEA)],
        out_specs=_row_spec(D, TILEA),
        out_shape=jax.ShapeDtypeStruct((M, D), jnp.float32),
        compiler_params=_PARALLEL,
        interpret=interpret,
    )(vm, X, R)

    emb = pl.pallas_call(
        _stage3_body,
        grid=(M // TILE,),
        in_specs=[_row_spec(D), _full_spec(D, D), _full_spec(1, D),
                  _full_spec(D, E), _full_spec(1, E)],
        out_specs=_row_spec(E),
        out_shape=jax.ShapeDtypeStruct((M, E), jnp.float32),
        compiler_params=_PARALLEL,
        interpret=interpret,
    )(x2, w3, b3, w4, b4)

    return emb.reshape(B, K, E)


def kernel(images, bboxes, img_range, gw1, gb1, gw2, gb2, nw1, nb1, nw2, nb2,
           mw1, mb1, mw2, mb2):
    return _run(images, bboxes, img_range, gw1, gb1, gw2, gb2, nw1, nb1,
                nw2, nb2, mw1, mb1, mw2, mb2)


# speedup vs baseline: 1.0436x; 1.0369x over previous
import functools

import jax
import jax.numpy as jnp
from jax.experimental import pallas as pl
from jax.experimental.pallas import tpu as pltpu

B, K, D, E, P = 128, 36, 2048, 1024, 5
M = B * K
TILE = 256
TILEA = 16 * K

_PARALLEL = pltpu.CompilerParams(dimension_semantics=("parallel",))


def _kernelA_body(x_ref, bb_ref, w1ig_ref, w1in_ref, w1eg_ref, w1en_ref,
                  gb1_ref, nb1_ref, w2n_ref, nb2_ref, gw2c_ref, gb2_ref,
                  vm_ref):
    X = x_ref[...].astype(jnp.bfloat16)
    bb = bb_ref[...]
    area = (bb[:, 2:3] - bb[:, 0:1]) * (bb[:, 3:4] - bb[:, 1:2])
    extras = (jnp.concatenate(
        [bb, area, jnp.zeros((TILE, 3), jnp.float32)], axis=1)
        * 0.1).astype(jnp.bfloat16)
    hg = jnp.dot(X, w1ig_ref[...], preferred_element_type=jnp.float32)
    hn = jnp.dot(X, w1in_ref[...], preferred_element_type=jnp.float32)
    hg = hg + jnp.dot(extras, w1eg_ref[...],
                      preferred_element_type=jnp.float32) + gb1_ref[...]
    hn = hn + jnp.dot(extras, w1en_ref[...],
                      preferred_element_type=jnp.float32) + nb1_ref[...]
    hg = jnp.maximum(hg, 0.0).astype(jnp.bfloat16)
    hn = jnp.maximum(hn, 0.0).astype(jnp.bfloat16)
    v = jnp.dot(hn, w2n_ref[...], preferred_element_type=jnp.float32)
    v = v + nb2_ref[...]
    gate = jnp.dot(hg, gw2c_ref[...],
                   preferred_element_type=jnp.float32)[:, 0:1] + gb2_ref[...]
    m = jax.nn.sigmoid(gate)
    vm_ref[...] = (m * v).astype(jnp.bfloat16)


def _kernelB_body(vm_ref, x_ref, r_ref, w3_ref, b3_ref, w4_ref, b4_ref,
                  o_ref):
    vm = vm_ref[...]
    R = r_ref[...]
    iu = jax.lax.broadcasted_iota(jnp.int32, (K, K), 0)
    ju = jax.lax.broadcasted_iota(jnp.int32, (K, K), 1)
    upper = (iu <= ju).astype(jnp.bfloat16)
    cs = jnp.dot(R.astype(jnp.bfloat16), upper,
                 preferred_element_type=jnp.float32)
    sel = jnp.where((R == 1.0) & (cs <= float(P)), 1.0, 0.0)
    deficit = float(P) - jnp.minimum(cs[:, K - 1:K], float(P))

    jg = jax.lax.broadcasted_iota(jnp.int32, (K, TILEA), 0)
    cg = jax.lax.broadcasted_iota(jnp.int32, (K, TILEA), 1)
    G = (cg % K == jg).astype(jnp.bfloat16)
    W = jnp.dot(sel.astype(jnp.bfloat16), G,
                preferred_element_type=jnp.float32)
    ri = jax.lax.broadcasted_iota(jnp.int32, (TILEA, TILEA), 0)
    ci = jax.lax.broadcasted_iota(jnp.int32, (TILEA, TILEA), 1)
    W = jnp.where((ri // K) == (ci // K), W, 0.0)
    W = W + jnp.where(ri == ci, deficit, 0.0)

    agg = jnp.dot(W.astype(jnp.bfloat16), vm,
                  preferred_element_type=jnp.float32)
    norm = jnp.sqrt(jnp.sum(agg * agg, axis=1, keepdims=True)) + 1e-8
    x2 = (x_ref[...] + agg / norm).astype(jnp.bfloat16)

    hm = jnp.dot(x2, w3_ref[...], preferred_element_type=jnp.float32)
    hm = jnp.maximum(hm + b3_ref[...], 0.0).astype(jnp.bfloat16)
    emb = jnp.dot(hm, w4_ref[...], preferred_element_type=jnp.float32)
    emb = emb + b4_ref[...]
    norm2 = jnp.sqrt(jnp.sum(emb * emb, axis=1, keepdims=True)) + 1e-8
    o_ref[...] = emb / norm2


def _prep_body(gw1_ref, nw1_ref, nw2_ref, mw1_ref, mw2_ref, gw2_ref,
               w1ig_ref, w1in_ref, w1eg_ref, w1en_ref, w2n_ref,
               w3_ref, w4_ref, gw2c_ref):
    w1ig_ref[...] = gw1_ref[:, :D].astype(jnp.bfloat16).T
    w1in_ref[...] = nw1_ref[:, :D].astype(jnp.bfloat16).T
    pad = jnp.zeros((3, gw1_ref.shape[0]), jnp.bfloat16)
    w1eg_ref[...] = jnp.concatenate(
        [gw1_ref[:, D:].astype(jnp.bfloat16).T, pad], axis=0)
    w1en_ref[...] = jnp.concatenate(
        [nw1_ref[:, D:].astype(jnp.bfloat16).T, pad], axis=0)
    w2n_ref[...] = nw2_ref[...].astype(jnp.bfloat16).T
    w3_ref[...] = mw1_ref[...].astype(jnp.bfloat16).T
    w4_ref[...] = mw2_ref[...].astype(jnp.bfloat16).T
    gw2c = gw2_ref[...].astype(jnp.bfloat16).T
    gw2c_ref[...] = jnp.concatenate(
        [gw2c, jnp.zeros((gw2c.shape[0], 127), jnp.bfloat16)], axis=1)


def _prep_weights(gw1, nw1, nw2, mw1, mw2, gw2, interpret):
    g = 8
    r1 = D // g
    r2 = E // g
    return pl.pallas_call(
        _prep_body,
        grid=(g,),
        in_specs=[pl.BlockSpec((r1, D + 5), lambda i: (i, 0)),
                  pl.BlockSpec((r1, D + 5), lambda i: (i, 0)),
                  pl.BlockSpec((r1, D), lambda i: (i, 0)),
                  pl.BlockSpec((r1, D), lambda i: (i, 0)),
                  pl.BlockSpec((r2, D), lambda i: (i, 0)),
                  pl.BlockSpec((1, r1), lambda i: (0, i))],
        out_specs=[pl.BlockSpec((D, r1), lambda i: (0, i)),
                   pl.BlockSpec((D, r1), lambda i: (0, i)),
                   pl.BlockSpec((8, r1), lambda i: (0, i)),
                   pl.BlockSpec((8, r1), lambda i: (0, i)),
                   pl.BlockSpec((D, r1), lambda i: (0, i)),
                   pl.BlockSpec((D, r1), lambda i: (0, i)),
                   pl.BlockSpec((D, r2), lambda i: (0, i)),
                   pl.BlockSpec((r1, 128), lambda i: (i, 0))],
        out_shape=[jax.ShapeDtypeStruct((D, D), jnp.bfloat16),
                   jax.ShapeDtypeStruct((D, D), jnp.bfloat16),
                   jax.ShapeDtypeStruct((8, D), jnp.bfloat16),
                   jax.ShapeDtypeStruct((8, D), jnp.bfloat16),
                   jax.ShapeDtypeStruct((D, D), jnp.bfloat16),
                   jax.ShapeDtypeStruct((D, D), jnp.bfloat16),
                   jax.ShapeDtypeStruct((D, E), jnp.bfloat16),
                   jax.ShapeDtypeStruct((D, 128), jnp.bfloat16)],
        compiler_params=_PARALLEL,
        interpret=interpret,
    )(gw1, nw1, nw2, mw1, mw2, gw2)


def _row_spec(n, t=TILE):
    return pl.BlockSpec((t, n), lambda i: (i, 0))


def _full_spec(m, n):
    return pl.BlockSpec((m, n), lambda i: (0, 0))


@functools.partial(jax.jit, static_argnames=("interpret",))
def _run(images, bboxes, img_range, gw1, gb1, gw2, gb2, nw1, nb1, nw2, nb2,
         mw1, mb1, mw2, mb2, interpret=False):
    X = images.reshape(M, D)
    bb = bboxes.reshape(M, 4)
    R = img_range.reshape(M, K)

    w1ig, w1in, w1eg, w1en, w2n, w3, w4, gw2c = _prep_weights(
        gw1, nw1, nw2, mw1, mw2, gw2, interpret)
    gb1r = gb1[None, :]
    nb1r = nb1[None, :]
    nb2r = nb2[None, :]
    gb2r = gb2[None, :]
    b3 = mb1[None, :]
    b4 = mb2[None, :]

    vm = pl.pallas_call(
        _kernelA_body,
        grid=(M // TILE,),
        in_specs=[_row_spec(D), _row_spec(4), _full_spec(D, D),
                  _full_spec(D, D), _full_spec(8, D), _full_spec(8, D),
                  _full_spec(1, D), _full_spec(1, D), _full_spec(D, D),
                  _full_spec(1, D), _full_spec(D, 128), _full_spec(1, 1)],
        out_specs=_row_spec(D),
        out_shape=jax.ShapeDtypeStruct((M, D), jnp.bfloat16),
        compiler_params=_PARALLEL,
        interpret=interpret,
    )(X, bb, w1ig, w1in, w1eg, w1en, gb1r, nb1r, w2n, nb2r, gw2c, gb2r)

    emb = pl.pallas_call(
        _kernelB_body,
        grid=(M // TILEA,),
        in_specs=[_row_spec(D, TILEA), _row_spec(D, TILEA),
                  _row_spec(K, TILEA), _full_spec(D, D), _full_spec(1, D),
                  _full_spec(D, E), _full_spec(1, E)],
        out_specs=_row_spec(E, TILEA),
        out_shape=jax.ShapeDtypeStruct((M, E), jnp.float32),
        compiler_params=_PARALLEL,
        interpret=interpret,
    )(vm, X, R, w3, b3, w4, b4)

    return emb.reshape(B, K, E)


def kernel(images, bboxes, img_range, gw1, gb1, gw2, gb2, nw1, nb1, nw2, nb2,
           mw1, mb1, mw2, mb2):
    return _run(images, bboxes, img_range, gw1, gb1, gw2, gb2, nw1, nb1,
                nw2, nb2, mw1, mb1, mw2, mb2)


# speedup vs baseline: 1.0610x; 1.0168x over previous
import functools

import jax
import jax.numpy as jnp
from jax.experimental import pallas as pl
from jax.experimental.pallas import tpu as pltpu

B, K, D, E, P = 128, 36, 2048, 1024, 5
M = B * K
TILE = 512
TILEA = 16 * K

_PARALLEL = pltpu.CompilerParams(dimension_semantics=("parallel",))


def _kernelA_body(x_ref, bb_ref, w1ig_ref, w1in_ref, w1eg_ref, w1en_ref,
                  gb1_ref, nb1_ref, w2n_ref, nb2_ref, gw2c_ref, gb2_ref,
                  vm_ref):
    X = x_ref[...].astype(jnp.bfloat16)
    bb = bb_ref[...]
    area = (bb[:, 2:3] - bb[:, 0:1]) * (bb[:, 3:4] - bb[:, 1:2])
    extras = (jnp.concatenate(
        [bb, area, jnp.zeros((TILE, 3), jnp.float32)], axis=1)
        * 0.1).astype(jnp.bfloat16)
    hg = jnp.dot(X, w1ig_ref[...], preferred_element_type=jnp.float32)
    hn = jnp.dot(X, w1in_ref[...], preferred_element_type=jnp.float32)
    hg = hg + jnp.dot(extras, w1eg_ref[...],
                      preferred_element_type=jnp.float32) + gb1_ref[...]
    hn = hn + jnp.dot(extras, w1en_ref[...],
                      preferred_element_type=jnp.float32) + nb1_ref[...]
    hg = jnp.maximum(hg, 0.0).astype(jnp.bfloat16)
    hn = jnp.maximum(hn, 0.0).astype(jnp.bfloat16)
    v = jnp.dot(hn, w2n_ref[...], preferred_element_type=jnp.float32)
    v = v + nb2_ref[...]
    gate = jnp.dot(hg, gw2c_ref[...],
                   preferred_element_type=jnp.float32)[:, 0:1] + gb2_ref[...]
    m = jax.nn.sigmoid(gate)
    vm_ref[...] = (m * v).astype(jnp.bfloat16)


def _kernelB_body(vm_ref, x_ref, r_ref, w3_ref, b3_ref, w4_ref, b4_ref,
                  o_ref):
    vm = vm_ref[...]
    R = r_ref[...]
    iu = jax.lax.broadcasted_iota(jnp.int32, (K, K), 0)
    ju = jax.lax.broadcasted_iota(jnp.int32, (K, K), 1)
    upper = (iu <= ju).astype(jnp.bfloat16)
    cs = jnp.dot(R.astype(jnp.bfloat16), upper,
                 preferred_element_type=jnp.float32)
    sel = jnp.where((R == 1.0) & (cs <= float(P)), 1.0, 0.0)
    deficit = float(P) - jnp.minimum(cs[:, K - 1:K], float(P))

    jg = jax.lax.broadcasted_iota(jnp.int32, (K, TILEA), 0)
    cg = jax.lax.broadcasted_iota(jnp.int32, (K, TILEA), 1)
    G = (cg % K == jg).astype(jnp.bfloat16)
    W = jnp.dot(sel.astype(jnp.bfloat16), G,
                preferred_element_type=jnp.float32)
    ri = jax.lax.broadcasted_iota(jnp.int32, (TILEA, TILEA), 0)
    ci = jax.lax.broadcasted_iota(jnp.int32, (TILEA, TILEA), 1)
    W = jnp.where((ri // K) == (ci // K), W, 0.0)
    W = W + jnp.where(ri == ci, deficit, 0.0)

    agg = jnp.dot(W.astype(jnp.bfloat16), vm,
                  preferred_element_type=jnp.float32)
    norm = jnp.sqrt(jnp.sum(agg * agg, axis=1, keepdims=True)) + 1e-8
    x2 = (x_ref[...] + agg / norm).astype(jnp.bfloat16)

    hm = jnp.dot(x2, w3_ref[...], preferred_element_type=jnp.float32)
    hm = jnp.maximum(hm + b3_ref[...], 0.0).astype(jnp.bfloat16)
    emb = jnp.dot(hm, w4_ref[...], preferred_element_type=jnp.float32)
    emb = emb + b4_ref[...]
    norm2 = jnp.sqrt(jnp.sum(emb * emb, axis=1, keepdims=True)) + 1e-8
    o_ref[...] = emb / norm2


def _prep_body(gw1_ref, nw1_ref, nw2_ref, mw1_ref, mw2_ref, gw2_ref,
               w1ig_ref, w1in_ref, w1eg_ref, w1en_ref, w2n_ref,
               w3_ref, w4_ref, gw2c_ref):
    w1ig_ref[...] = gw1_ref[:, :D].astype(jnp.bfloat16).T
    w1in_ref[...] = nw1_ref[:, :D].astype(jnp.bfloat16).T
    pad = jnp.zeros((3, gw1_ref.shape[0]), jnp.bfloat16)
    w1eg_ref[...] = jnp.concatenate(
        [gw1_ref[:, D:].astype(jnp.bfloat16).T, pad], axis=0)
    w1en_ref[...] = jnp.concatenate(
        [nw1_ref[:, D:].astype(jnp.bfloat16).T, pad], axis=0)
    w2n_ref[...] = nw2_ref[...].astype(jnp.bfloat16).T
    w3_ref[...] = mw1_ref[...].astype(jnp.bfloat16).T
    w4_ref[...] = mw2_ref[...].astype(jnp.bfloat16).T
    gw2c = gw2_ref[...].astype(jnp.bfloat16).T
    gw2c_ref[...] = jnp.concatenate(
        [gw2c, jnp.zeros((gw2c.shape[0], 127), jnp.bfloat16)], axis=1)


def _prep_weights(gw1, nw1, nw2, mw1, mw2, gw2, interpret):
    g = 8
    r1 = D // g
    r2 = E // g
    return pl.pallas_call(
        _prep_body,
        grid=(g,),
        in_specs=[pl.BlockSpec((r1, D + 5), lambda i: (i, 0)),
                  pl.BlockSpec((r1, D + 5), lambda i: (i, 0)),
                  pl.BlockSpec((r1, D), lambda i: (i, 0)),
                  pl.BlockSpec((r1, D), lambda i: (i, 0)),
                  pl.BlockSpec((r2, D), lambda i: (i, 0)),
                  pl.BlockSpec((1, r1), lambda i: (0, i))],
        out_specs=[pl.BlockSpec((D, r1), lambda i: (0, i)),
                   pl.BlockSpec((D, r1), lambda i: (0, i)),
                   pl.BlockSpec((8, r1), lambda i: (0, i)),
                   pl.BlockSpec((8, r1), lambda i: (0, i)),
                   pl.BlockSpec((D, r1), lambda i: (0, i)),
                   pl.BlockSpec((D, r1), lambda i: (0, i)),
                   pl.BlockSpec((D, r2), lambda i: (0, i)),
                   pl.BlockSpec((r1, 128), lambda i: (i, 0))],
        out_shape=[jax.ShapeDtypeStruct((D, D), jnp.bfloat16),
                   jax.ShapeDtypeStruct((D, D), jnp.bfloat16),
                   jax.ShapeDtypeStruct((8, D), jnp.bfloat16),
                   jax.ShapeDtypeStruct((8, D), jnp.bfloat16),
                   jax.ShapeDtypeStruct((D, D), jnp.bfloat16),
                   jax.ShapeDtypeStruct((D, D), jnp.bfloat16),
                   jax.ShapeDtypeStruct((D, E), jnp.bfloat16),
                   jax.ShapeDtypeStruct((D, 128), jnp.bfloat16)],
        compiler_params=_PARALLEL,
        interpret=interpret,
    )(gw1, nw1, nw2, mw1, mw2, gw2)


def _row_spec(n, t=TILE):
    return pl.BlockSpec((t, n), lambda i: (i, 0))


def _full_spec(m, n):
    return pl.BlockSpec((m, n), lambda i: (0, 0))


@functools.partial(jax.jit, static_argnames=("interpret",))
def _run(images, bboxes, img_range, gw1, gb1, gw2, gb2, nw1, nb1, nw2, nb2,
         mw1, mb1, mw2, mb2, interpret=False):
    X = images.reshape(M, D)
    bb = bboxes.reshape(M, 4)
    R = img_range.reshape(M, K)

    w1ig, w1in, w1eg, w1en, w2n, w3, w4, gw2c = _prep_weights(
        gw1, nw1, nw2, mw1, mw2, gw2, interpret)
    gb1r = gb1[None, :]
    nb1r = nb1[None, :]
    nb2r = nb2[None, :]
    gb2r = gb2[None, :]
    b3 = mb1[None, :]
    b4 = mb2[None, :]

    vm = pl.pallas_call(
        _kernelA_body,
        grid=(M // TILE,),
        in_specs=[_row_spec(D), _row_spec(4), _full_spec(D, D),
                  _full_spec(D, D), _full_spec(8, D), _full_spec(8, D),
                  _full_spec(1, D), _full_spec(1, D), _full_spec(D, D),
                  _full_spec(1, D), _full_spec(D, 128), _full_spec(1, 1)],
        out_specs=_row_spec(D),
        out_shape=jax.ShapeDtypeStruct((M, D), jnp.bfloat16),
        compiler_params=_PARALLEL,
        interpret=interpret,
    )(X, bb, w1ig, w1in, w1eg, w1en, gb1r, nb1r, w2n, nb2r, gw2c, gb2r)

    emb = pl.pallas_call(
        _kernelB_body,
        grid=(M // TILEA,),
        in_specs=[_row_spec(D, TILEA), _row_spec(D, TILEA),
                  _row_spec(K, TILEA), _full_spec(D, D), _full_spec(1, D),
                  _full_spec(D, E), _full_spec(1, E)],
        out_specs=_row_spec(E, TILEA),
        out_shape=jax.ShapeDtypeStruct((M, E), jnp.float32),
        compiler_params=_PARALLEL,
        interpret=interpret,
    )(vm, X, R, w3, b3, w4, b4)

    return emb.reshape(B, K, E)


def kernel(images, bboxes, img_range, gw1, gb1, gw2, gb2, nw1, nb1, nw2, nb2,
           mw1, mb1, mw2, mb2):
    return _run(images, bboxes, img_range, gw1, gb1, gw2, gb2, nw1, nb1,
                nw2, nb2, mw1, mb1, mw2, mb2)
